# Initial kernel scaffold; baseline (speedup 1.0000x reference)
import jax
import jax.numpy as jnp
from jax.experimental import pallas as pl


def kernel(x, Wg, w1, w2):
    raise NotImplementedError("write your pallas kernel here")


# speedup vs baseline: 4.3700x; 4.3700x over previous
import jax
import jax.numpy as jnp
from jax.experimental import pallas as pl
from jax.experimental.pallas import tpu as pltpu

B, S, D, E, H, TOPK = 2, 2048, 768, 8, 768, 2
AUX_COEF = 0.01
N = B * S
T = 512
NT = N // T


def _moe_body(x_ref, wg_ref, w1_ref, w2_ref, out_ref, aux_ref,
              wts_ref, accf_ref, accp_ref):
    t = pl.program_id(0)
    e = pl.program_id(1)

    @pl.when(e == 0)
    def _router():
        xb = x_ref[...]
        logits = jax.lax.dot_general(
            xb, wg_ref[...], (((1,), (1,)), ((), ())),
            preferred_element_type=jnp.float32)
        m = jnp.max(logits, axis=-1, keepdims=True)
        ex = jnp.exp(logits - m)
        probs = ex / jnp.sum(ex, axis=-1, keepdims=True)
        ids = jax.lax.broadcasted_iota(jnp.int32, (T, E), 1)
        m1 = jnp.max(probs, axis=-1, keepdims=True)
        i1 = jnp.min(jnp.where(probs == m1, ids, E), axis=-1, keepdims=True)
        oh1 = (ids == i1)
        probs2 = jnp.where(oh1, -jnp.inf, probs)
        m2 = jnp.max(probs2, axis=-1, keepdims=True)
        i2 = jnp.min(jnp.where(probs2 == m2, ids, E), axis=-1, keepdims=True)
        sel = (oh1 | (ids == i2)).astype(jnp.float32)
        pw = probs * sel
        wts_ref[...] = pw / jnp.sum(pw, axis=-1, keepdims=True)

        @pl.when(t == 0)
        def _init():
            accf_ref[...] = jnp.zeros_like(accf_ref)
            accp_ref[...] = jnp.zeros_like(accp_ref)

        accf_ref[...] += jnp.sum(oh1.astype(jnp.float32), axis=0,
                                 keepdims=True)
        accp_ref[...] += jnp.sum(probs, axis=0, keepdims=True)

    xb = x_ref[...]
    h = jnp.maximum(
        jax.lax.dot_general(xb, w1_ref[0], (((1,), (0,)), ((), ())),
                            preferred_element_type=jnp.float32), 0.0)
    o = jax.lax.dot_general(h, w2_ref[0], (((1,), (0,)), ((), ())),
                            preferred_element_type=jnp.float32)
    wts = wts_ref[...]
    lane = jax.lax.broadcasted_iota(jnp.int32, (T, E), 1)
    w_e = jnp.sum(jnp.where(lane == e, wts, 0.0), axis=1, keepdims=True)
    contrib = w_e * o

    @pl.when(e == 0)
    def _first():
        out_ref[...] = contrib

    @pl.when(e != 0)
    def _acc():
        out_ref[...] += contrib

    @pl.when((t == NT - 1) & (e == E - 1))
    def _aux():
        f = accf_ref[0] / N
        P = accp_ref[0] / N
        aux_ref[...] = (AUX_COEF * E * jnp.sum(f * P)).reshape(1, 1)


def kernel(x, Wg, w1, w2):
    x_flat = x.reshape(N, D)
    out, aux = pl.pallas_call(
        _moe_body,
        grid=(NT, E),
        in_specs=[
            pl.BlockSpec((T, D), lambda t, e: (t, 0)),
            pl.BlockSpec((E, D), lambda t, e: (0, 0)),
            pl.BlockSpec((1, D, H), lambda t, e: (e, 0, 0)),
            pl.BlockSpec((1, H, D), lambda t, e: (e, 0, 0)),
        ],
        out_specs=[
            pl.BlockSpec((T, D), lambda t, e: (t, 0)),
            pl.BlockSpec((1, 1), lambda t, e: (0, 0)),
        ],
        out_shape=[
            jax.ShapeDtypeStruct((N, D), jnp.float32),
            jax.ShapeDtypeStruct((1, 1), jnp.float32),
        ],
        scratch_shapes=[
            pltpu.VMEM((T, E), jnp.float32),
            pltpu.VMEM((1, E), jnp.float32),
            pltpu.VMEM((1, E), jnp.float32),
        ],
        compiler_params=pltpu.CompilerParams(
            dimension_semantics=("arbitrary", "arbitrary")),
    )(x_flat, Wg, w1, w2)
    return out.reshape(B, S, D), aux[0, 0]


# speedup vs baseline: 4.3922x; 1.0051x over previous
import jax
import jax.numpy as jnp
from jax.experimental import pallas as pl
from jax.experimental.pallas import tpu as pltpu

B, S, D, E, H, TOPK = 2, 2048, 768, 8, 768, 2
AUX_COEF = 0.01
N = B * S
T = 512
NT = N // T


def _moe_body(x_ref, wg_ref, w1_ref, w2_ref, out_ref, aux_ref,
              wts_ref, accf_ref, accp_ref):
    t = pl.program_id(0)
    e = pl.program_id(1)

    @pl.when(e == 0)
    def _router():
        xb = x_ref[...]
        logits = jax.lax.dot_general(
            xb, wg_ref[...], (((1,), (1,)), ((), ())),
            preferred_element_type=jnp.float32)
        m = jnp.max(logits, axis=-1, keepdims=True)
        ex = jnp.exp(logits - m)
        probs = ex / jnp.sum(ex, axis=-1, keepdims=True)
        ids = jax.lax.broadcasted_iota(jnp.int32, (T, E), 1)
        m1 = jnp.max(probs, axis=-1, keepdims=True)
        i1 = jnp.min(jnp.where(probs == m1, ids, E), axis=-1, keepdims=True)
        oh1 = (ids == i1)
        probs2 = jnp.where(oh1, -jnp.inf, probs)
        m2 = jnp.max(probs2, axis=-1, keepdims=True)
        i2 = jnp.min(jnp.where(probs2 == m2, ids, E), axis=-1, keepdims=True)
        sel = (oh1 | (ids == i2)).astype(jnp.float32)
        pw = probs * sel
        wts_ref[...] = pw / jnp.sum(pw, axis=-1, keepdims=True)

        @pl.when(t == 0)
        def _init():
            accf_ref[...] = jnp.zeros_like(accf_ref)
            accp_ref[...] = jnp.zeros_like(accp_ref)

        accf_ref[...] += jnp.sum(oh1.astype(jnp.float32), axis=0,
                                 keepdims=True)
        accp_ref[...] += jnp.sum(probs, axis=0, keepdims=True)

    xb = x_ref[...].astype(jnp.bfloat16)
    h = jnp.maximum(
        jax.lax.dot_general(xb, w1_ref[0].astype(jnp.bfloat16),
                            (((1,), (0,)), ((), ())),
                            preferred_element_type=jnp.float32), 0.0)
    o = jax.lax.dot_general(h.astype(jnp.bfloat16),
                            w2_ref[0].astype(jnp.bfloat16),
                            (((1,), (0,)), ((), ())),
                            preferred_element_type=jnp.float32)
    wts = wts_ref[...]
    lane = jax.lax.broadcasted_iota(jnp.int32, (T, E), 1)
    w_e = jnp.sum(jnp.where(lane == e, wts, 0.0), axis=1, keepdims=True)
    contrib = w_e * o

    @pl.when(e == 0)
    def _first():
        out_ref[...] = contrib

    @pl.when(e != 0)
    def _acc():
        out_ref[...] += contrib

    @pl.when((t == NT - 1) & (e == E - 1))
    def _aux():
        f = accf_ref[0] / N
        P = accp_ref[0] / N
        aux_ref[...] = (AUX_COEF * E * jnp.sum(f * P)).reshape(1, 1)


def kernel(x, Wg, w1, w2):
    x_flat = x.reshape(N, D)
    out, aux = pl.pallas_call(
        _moe_body,
        grid=(NT, E),
        in_specs=[
            pl.BlockSpec((T, D), lambda t, e: (t, 0)),
            pl.BlockSpec((E, D), lambda t, e: (0, 0)),
            pl.BlockSpec((1, D, H), lambda t, e: (e, 0, 0)),
            pl.BlockSpec((1, H, D), lambda t, e: (e, 0, 0)),
        ],
        out_specs=[
            pl.BlockSpec((T, D), lambda t, e: (t, 0)),
            pl.BlockSpec((1, 1), lambda t, e: (0, 0)),
        ],
        out_shape=[
            jax.ShapeDtypeStruct((N, D), jnp.float32),
            jax.ShapeDtypeStruct((1, 1), jnp.float32),
        ],
        scratch_shapes=[
            pltpu.VMEM((T, E), jnp.float32),
            pltpu.VMEM((1, E), jnp.float32),
            pltpu.VMEM((1, E), jnp.float32),
        ],
        compiler_params=pltpu.CompilerParams(
            dimension_semantics=("arbitrary", "arbitrary")),
    )(x_flat, Wg, w1, w2)
    return out.reshape(B, S, D), aux[0, 0]


# speedup vs baseline: 6.0064x; 1.3675x over previous
import jax
import jax.numpy as jnp
from jax.experimental import pallas as pl
from jax.experimental.pallas import tpu as pltpu

B, S, D, E, H, TOPK = 2, 2048, 768, 8, 768, 2
AUX_COEF = 0.01
N = B * S
T = 2048
NT = N // T


def _moe_body(x_ref, wg_ref, w1_ref, w2_ref, out_ref, aux_ref,
              wts_ref, accf_ref, accp_ref):
    t = pl.program_id(0)
    e = pl.program_id(1)

    @pl.when(e == 0)
    def _router():
        xb = x_ref[...]
        logits = jax.lax.dot_general(
            xb, wg_ref[...], (((1,), (1,)), ((), ())),
            preferred_element_type=jnp.float32)
        m = jnp.max(logits, axis=-1, keepdims=True)
        ex = jnp.exp(logits - m)
        probs = ex / jnp.sum(ex, axis=-1, keepdims=True)
        ids = jax.lax.broadcasted_iota(jnp.int32, (T, E), 1)
        m1 = jnp.max(probs, axis=-1, keepdims=True)
        i1 = jnp.min(jnp.where(probs == m1, ids, E), axis=-1, keepdims=True)
        oh1 = (ids == i1)
        probs2 = jnp.where(oh1, -jnp.inf, probs)
        m2 = jnp.max(probs2, axis=-1, keepdims=True)
        i2 = jnp.min(jnp.where(probs2 == m2, ids, E), axis=-1, keepdims=True)
        sel = (oh1 | (ids == i2)).astype(jnp.float32)
        pw = probs * sel
        wts_ref[...] = pw / jnp.sum(pw, axis=-1, keepdims=True)

        @pl.when(t == 0)
        def _init():
            accf_ref[...] = jnp.zeros_like(accf_ref)
            accp_ref[...] = jnp.zeros_like(accp_ref)

        accf_ref[...] += jnp.sum(oh1.astype(jnp.float32), axis=0,
                                 keepdims=True)
        accp_ref[...] += jnp.sum(probs, axis=0, keepdims=True)

    xb = x_ref[...].astype(jnp.bfloat16)
    h = jnp.maximum(
        jax.lax.dot_general(xb, w1_ref[0].astype(jnp.bfloat16),
                            (((1,), (0,)), ((), ())),
                            preferred_element_type=jnp.float32), 0.0)
    o = jax.lax.dot_general(h.astype(jnp.bfloat16),
                            w2_ref[0].astype(jnp.bfloat16),
                            (((1,), (0,)), ((), ())),
                            preferred_element_type=jnp.float32)
    wts = wts_ref[...]
    lane = jax.lax.broadcasted_iota(jnp.int32, (T, E), 1)
    w_e = jnp.sum(jnp.where(lane == e, wts, 0.0), axis=1, keepdims=True)
    contrib = w_e * o

    @pl.when(e == 0)
    def _first():
        out_ref[...] = contrib

    @pl.when(e != 0)
    def _acc():
        out_ref[...] += contrib

    @pl.when((t == NT - 1) & (e == E - 1))
    def _aux():
        f = accf_ref[0] / N
        P = accp_ref[0] / N
        aux_ref[...] = (AUX_COEF * E * jnp.sum(f * P)).reshape(1, 1)


def kernel(x, Wg, w1, w2):
    x_flat = x.reshape(N, D)
    out, aux = pl.pallas_call(
        _moe_body,
        grid=(NT, E),
        in_specs=[
            pl.BlockSpec((T, D), lambda t, e: (t, 0)),
            pl.BlockSpec((E, D), lambda t, e: (0, 0)),
            pl.BlockSpec((1, D, H), lambda t, e: (e, 0, 0)),
            pl.BlockSpec((1, H, D), lambda t, e: (e, 0, 0)),
        ],
        out_specs=[
            pl.BlockSpec((T, D), lambda t, e: (t, 0)),
            pl.BlockSpec((1, 1), lambda t, e: (0, 0)),
        ],
        out_shape=[
            jax.ShapeDtypeStruct((N, D), jnp.float32),
            jax.ShapeDtypeStruct((1, 1), jnp.float32),
        ],
        scratch_shapes=[
            pltpu.VMEM((T, E), jnp.float32),
            pltpu.VMEM((1, E), jnp.float32),
            pltpu.VMEM((1, E), jnp.float32),
        ],
        compiler_params=pltpu.CompilerParams(
            dimension_semantics=("arbitrary", "arbitrary")),
    )(x_flat, Wg, w1, w2)
    return out.reshape(B, S, D), aux[0, 0]
